# Initial kernel scaffold; baseline (speedup 1.0000x reference)
import jax
import jax.numpy as jnp
from jax.experimental import pallas as pl


def kernel(x, W_enc, b_enc, W_dec, b_dec):
    raise NotImplementedError("write your pallas kernel here")


# speedup vs baseline: 4.6285x; 4.6285x over previous
import functools

import jax
import jax.numpy as jnp
from jax import lax
from jax.experimental import pallas as pl
from jax.experimental.pallas import tpu as pltpu
from jax.experimental.pallas import tpu_sc as plsc

D_IN = 2048
NL = 32768
K = 32
B = 1024

SEG = 128
NSEG = NL // SEG
LBLK = 512
NLB = NL // LBLK
SEG_PER_BLK = LBLK // SEG
NCAND = K * SEG
PADW = 128

_NEG = -3.0e38


def _encode_body(x_ref, w_ref, benc_ref, bdec_ref, pre_ref, m_ref):
    xb = (x_ref[...] - bdec_ref[...]).astype(jnp.bfloat16)
    wb = w_ref[...].astype(jnp.bfloat16)
    pre = lax.dot_general(xb, wb, (((1,), (1,)), ((), ())),
                          preferred_element_type=jnp.float32)
    pre = pre + benc_ref[...].reshape(1, LBLK)
    pre_ref[...] = pre
    m_ref[...] = jnp.max(pre.reshape(B, SEG_PER_BLK, SEG), axis=-1)[None]


def _encode(x, W_enc, b_enc, b_dec):
    return pl.pallas_call(
        _encode_body,
        grid=(NLB,),
        in_specs=[
            pl.BlockSpec((B, D_IN), lambda i: (0, 0)),
            pl.BlockSpec((LBLK, D_IN), lambda i: (i, 0)),
            pl.BlockSpec((1, 1, LBLK), lambda i: (i, 0, 0)),
            pl.BlockSpec((1, D_IN), lambda i: (0, 0)),
        ],
        out_specs=[
            pl.BlockSpec((B, LBLK), lambda i: (0, i)),
            pl.BlockSpec((1, B, SEG_PER_BLK), lambda i: (i, 0, 0)),
        ],
        out_shape=[
            jax.ShapeDtypeStruct((B, NL), jnp.float32),
            jax.ShapeDtypeStruct((NLB, B, SEG_PER_BLK), jnp.float32),
        ],
    )(x, W_enc, b_enc.reshape(NLB, 1, LBLK), b_dec.reshape(1, D_IN))


def _topseg_body(m_ref, seg_ref):
    iota = lax.broadcasted_iota(jnp.int32, (B, NSEG), 1)
    lane = lax.broadcasted_iota(jnp.int32, (B, PADW), 1)

    def step(k, carry):
        m, out = carry
        row_max = jnp.max(m, axis=1, keepdims=True)
        pos = jnp.min(jnp.where(m == row_max, iota, NSEG), axis=1,
                      keepdims=True)
        out = jnp.where(lane == k, pos, out)
        m = jnp.where(iota == pos, _NEG, m)
        return m, out

    _, out = lax.fori_loop(0, K, step,
                           (m_ref[...], jnp.zeros((B, PADW), jnp.int32)))
    seg_ref[...] = out


def _topseg(M):
    return pl.pallas_call(
        _topseg_body,
        out_shape=jax.ShapeDtypeStruct((B, PADW), jnp.int32),
    )(M)


def _gather_cand(pre_flat, seg_idx):
    mesh = plsc.VectorSubcoreMesh(core_axis_name="c", subcore_axis_name="s")
    info = plsc.get_sparse_core_info()
    nw = info.num_cores * info.num_subcores
    rows_per_w = B // nw

    @functools.partial(
        pl.kernel,
        mesh=mesh,
        out_type=jax.ShapeDtypeStruct((B, K, SEG), jnp.float32),
        scratch_types=[
            pltpu.VMEM((rows_per_w, PADW), jnp.int32),
            pltpu.VMEM((K,), jnp.int32),
            pltpu.VMEM((K, SEG), jnp.float32),
            pltpu.SemaphoreType.DMA,
        ],
    )
    def k3(pre_hbm, seg_hbm, cand_hbm, segs_v, idx_v, cand_v, sem):
        wid = lax.axis_index("s") * info.num_cores + lax.axis_index("c")
        base = wid * rows_per_w
        pltpu.sync_copy(seg_hbm.at[pl.ds(base, rows_per_w)], segs_v)

        def row_body(r, carry):
            row = base + r
            for c in range(K // 16):
                segs = segs_v[r, pl.ds(c * 16, 16)]
                idx_v[pl.ds(c * 16, 16)] = segs + row * NSEG
            pltpu.async_copy(pre_hbm.at[idx_v], cand_v, sem).wait()
            pltpu.sync_copy(cand_v, cand_hbm.at[row])
            return carry

        lax.fori_loop(0, rows_per_w, row_body, 0)

    return k3(pre_flat, seg_idx)


BBLK = 128


def _topcand_body(c_ref, act_ref, pos_ref):
    iota = lax.broadcasted_iota(jnp.int32, (BBLK, NCAND), 1)
    lane = lax.broadcasted_iota(jnp.int32, (BBLK, PADW), 1)

    def step(k, carry):
        c, acts, poss = carry
        row_max = jnp.max(c, axis=1, keepdims=True)
        pos = jnp.min(jnp.where(c == row_max, iota, NCAND), axis=1,
                      keepdims=True)
        acts = jnp.where(lane == k, row_max, acts)
        poss = jnp.where(lane == k, pos, poss)
        c = jnp.where(iota == pos, _NEG, c)
        return c, acts, poss

    _, acts, poss = lax.fori_loop(
        0, K, step,
        (c_ref[...], jnp.zeros((BBLK, PADW), jnp.float32),
         jnp.zeros((BBLK, PADW), jnp.int32)))
    act_ref[...] = acts
    pos_ref[...] = poss


def _topcand(cand):
    return pl.pallas_call(
        _topcand_body,
        grid=(B // BBLK,),
        in_specs=[pl.BlockSpec((BBLK, NCAND), lambda i: (i, 0))],
        out_specs=[
            pl.BlockSpec((BBLK, PADW), lambda i: (i, 0)),
            pl.BlockSpec((BBLK, PADW), lambda i: (i, 0)),
        ],
        out_shape=[
            jax.ShapeDtypeStruct((B, PADW), jnp.float32),
            jax.ShapeDtypeStruct((B, PADW), jnp.int32),
        ],
    )(cand)


def _decode(W_dec, b_dec, seg_idx, acts, pos):
    mesh = plsc.VectorSubcoreMesh(core_axis_name="c", subcore_axis_name="s")
    info = plsc.get_sparse_core_info()
    nw = info.num_cores * info.num_subcores
    rows_per_w = B // nw
    HK = 16

    @functools.partial(
        pl.kernel,
        mesh=mesh,
        out_type=jax.ShapeDtypeStruct((B, D_IN), jnp.float32),
        scratch_types=[
            pltpu.VMEM((rows_per_w * PADW,), jnp.int32),
            pltpu.VMEM((rows_per_w * PADW,), jnp.int32),
            pltpu.VMEM((rows_per_w * PADW,), jnp.float32),
            pltpu.VMEM((D_IN,), jnp.float32),
            pltpu.VMEM((D_IN,), jnp.float32),
            pltpu.VMEM((HK, D_IN), jnp.float32),
            pltpu.VMEM((HK, D_IN), jnp.float32),
            pltpu.SemaphoreType.DMA,
            pltpu.SemaphoreType.DMA,
        ],
    )
    def k5(wdec_hbm, bdec_hbm, seg_hbm, act_hbm, pos_hbm, out_hbm,
           segs_v, pos_v, acts_v, bdec_v, acc_v, buf_a, buf_b, sem_a, sem_b):
        wid = lax.axis_index("s") * info.num_cores + lax.axis_index("c")
        base = wid * rows_per_w
        nflat = rows_per_w * PADW
        pltpu.sync_copy(seg_hbm.at[pl.ds(base * PADW, nflat)], segs_v)
        pltpu.sync_copy(pos_hbm.at[pl.ds(base * PADW, nflat)], pos_v)
        pltpu.sync_copy(act_hbm.at[pl.ds(base * PADW, nflat)], acts_v)
        pltpu.sync_copy(bdec_hbm, bdec_v)

        iota16 = lax.iota(jnp.int32, 16)

        def dyn_gather16(vec, idx):
            return lax.gather(
                vec, idx[:, None],
                lax.GatherDimensionNumbers(
                    offset_dims=(), collapsed_slice_dims=(0,),
                    start_index_map=(0,)),
                (1,), mode=lax.GatherScatterMode.PROMISE_IN_BOUNDS)

        def lat_for(r, h):
            p = pos_v[pl.ds(pl.multiple_of(r * PADW + h * 16, 16), 16)]
            sel = p >> 7
            seg_lo = segs_v[pl.ds(pl.multiple_of(r * PADW, 16), 16)]
            seg_hi = segs_v[pl.ds(pl.multiple_of(r * PADW + 16, 16), 16)]
            low = sel & 15
            g_lo = dyn_gather16(seg_lo, low)
            g_hi = dyn_gather16(seg_hi, low)
            s = jnp.where(sel < 16, g_lo, g_hi)
            return s * SEG + (p & 127)

        def accumulate(r, h, buf, init_from_bdec):
            av = acts_v[pl.ds(pl.multiple_of(r * PADW + h * 16, 16), 16)]
            a = [dyn_gather16(av, jnp.full((16,), k, jnp.int32))
                 for k in range(HK)]

            def chunk(c, carry):
                ds = pl.ds(pl.multiple_of(c * 16, 16), 16)
                if init_from_bdec:
                    v = bdec_v[ds]
                else:
                    v = acc_v[ds]
                for k in range(HK):
                    v = v + a[k] * buf[k, ds]
                acc_v[ds] = v
                return carry

            lax.fori_loop(0, D_IN // 16, chunk, 0)

        def wait_a():
            pltpu.make_async_copy(wdec_hbm.at[pl.ds(0, HK)], buf_a,
                                  sem_a).wait()

        def row_body(r, carry):
            row = base + r
            cp_b = pltpu.async_copy(wdec_hbm.at[lat_for(r, 1)], buf_b, sem_b)
            wait_a()
            accumulate(r, 0, buf_a, True)
            r_next = jnp.minimum(r + 1, rows_per_w - 1)
            pltpu.async_copy(wdec_hbm.at[lat_for(r_next, 0)], buf_a, sem_a)
            cp_b.wait()
            accumulate(r, 1, buf_b, False)
            pltpu.sync_copy(acc_v, out_hbm.at[row])
            return carry

        pltpu.async_copy(wdec_hbm.at[lat_for(0, 0)], buf_a, sem_a)
        lax.fori_loop(0, rows_per_w, row_body, 0)
        wait_a()

    return k5(W_dec, b_dec, seg_idx, acts, pos)


def kernel(x, W_enc, b_enc, W_dec, b_dec):
    pre, M3 = _encode(x, W_enc, b_enc, b_dec)
    seg_idx = _topseg(M3.transpose(1, 0, 2).reshape(B, NSEG))
    cand = _gather_cand(pre.reshape(B * NSEG, SEG), seg_idx)
    acts, pos = _topcand(cand.reshape(B, NCAND))
    return _decode(W_dec, b_dec, seg_idx.reshape(-1), acts.reshape(-1),
                   pos.reshape(-1))


# speedup vs baseline: 5.0871x; 1.0991x over previous
import functools

import jax
import jax.numpy as jnp
from jax import lax
from jax.experimental import pallas as pl
from jax.experimental.pallas import tpu as pltpu
from jax.experimental.pallas import tpu_sc as plsc

D_IN = 2048
NL = 32768
K = 32
B = 1024

SEG = 128
NSEG = NL // SEG
LBLK = 1024
NLB = NL // LBLK
SEG_PER_BLK = LBLK // SEG
NCAND = K * SEG
PADW = 128

_NEG = -3.0e38


def _encode_body(x_ref, w_ref, benc_ref, bdec_ref, pre_ref, m_ref):
    xb = (x_ref[...] - bdec_ref[...]).astype(jnp.bfloat16)
    wb = w_ref[...].astype(jnp.bfloat16)
    pre = lax.dot_general(xb, wb, (((1,), (1,)), ((), ())),
                          preferred_element_type=jnp.float32)
    pre = pre + benc_ref[...].reshape(1, LBLK)
    pre_ref[...] = pre
    m_ref[...] = jnp.max(pre.reshape(B, SEG_PER_BLK, SEG), axis=-1)[None]


def _encode(x, W_enc, b_enc, b_dec):
    return pl.pallas_call(
        _encode_body,
        grid=(NLB,),
        in_specs=[
            pl.BlockSpec((B, D_IN), lambda i: (0, 0)),
            pl.BlockSpec((LBLK, D_IN), lambda i: (i, 0)),
            pl.BlockSpec((1, 1, LBLK), lambda i: (i, 0, 0)),
            pl.BlockSpec((1, D_IN), lambda i: (0, 0)),
        ],
        out_specs=[
            pl.BlockSpec((B, LBLK), lambda i: (0, i)),
            pl.BlockSpec((1, B, SEG_PER_BLK), lambda i: (i, 0, 0)),
        ],
        out_shape=[
            jax.ShapeDtypeStruct((B, NL), jnp.float32),
            jax.ShapeDtypeStruct((NLB, B, SEG_PER_BLK), jnp.float32),
        ],
    )(x, W_enc, b_enc.reshape(NLB, 1, LBLK), b_dec.reshape(1, D_IN))


def _topseg_body(m_ref, seg_ref):
    iota = lax.broadcasted_iota(jnp.int32, (B, NSEG), 1)
    lane = lax.broadcasted_iota(jnp.int32, (B, PADW), 1)

    def step(k, carry):
        m, out = carry
        row_max = jnp.max(m, axis=1, keepdims=True)
        pos = jnp.min(jnp.where(m == row_max, iota, NSEG), axis=1,
                      keepdims=True)
        out = jnp.where(lane == k, pos, out)
        m = jnp.where(iota == pos, _NEG, m)
        return m, out

    _, out = lax.fori_loop(0, K, step,
                           (m_ref[...], jnp.zeros((B, PADW), jnp.int32)))
    seg_ref[...] = out


def _topseg(M):
    return pl.pallas_call(
        _topseg_body,
        out_shape=jax.ShapeDtypeStruct((B, PADW), jnp.int32),
    )(M)


def _gather_cand(pre_flat, seg_idx):
    mesh = plsc.VectorSubcoreMesh(core_axis_name="c", subcore_axis_name="s")
    info = plsc.get_sparse_core_info()
    nw = info.num_cores * info.num_subcores
    rows_per_w = B // nw

    @functools.partial(
        pl.kernel,
        mesh=mesh,
        out_type=jax.ShapeDtypeStruct((B, K, SEG), jnp.float32),
        scratch_types=[
            pltpu.VMEM((rows_per_w, PADW), jnp.int32),
            pltpu.VMEM((K,), jnp.int32),
            pltpu.VMEM((K, SEG), jnp.float32),
            pltpu.SemaphoreType.DMA,
        ],
    )
    def k3(pre_hbm, seg_hbm, cand_hbm, segs_v, idx_v, cand_v, sem):
        wid = lax.axis_index("s") * info.num_cores + lax.axis_index("c")
        base = wid * rows_per_w
        pltpu.sync_copy(seg_hbm.at[pl.ds(base, rows_per_w)], segs_v)

        def row_body(r, carry):
            row = base + r
            for c in range(K // 16):
                segs = segs_v[r, pl.ds(c * 16, 16)]
                idx_v[pl.ds(c * 16, 16)] = segs + row * NSEG
            pltpu.async_copy(pre_hbm.at[idx_v], cand_v, sem).wait()
            pltpu.sync_copy(cand_v, cand_hbm.at[row])
            return carry

        lax.fori_loop(0, rows_per_w, row_body, 0)

    return k3(pre_flat, seg_idx)


BBLK = 128


def _topcand_body(c_ref, act_ref, pos_ref):
    iota = lax.broadcasted_iota(jnp.int32, (BBLK, NCAND), 1)
    lane = lax.broadcasted_iota(jnp.int32, (BBLK, PADW), 1)

    def step(k, carry):
        c, acts, poss = carry
        row_max = jnp.max(c, axis=1, keepdims=True)
        pos = jnp.min(jnp.where(c == row_max, iota, NCAND), axis=1,
                      keepdims=True)
        acts = jnp.where(lane == k, row_max, acts)
        poss = jnp.where(lane == k, pos, poss)
        c = jnp.where(iota == pos, _NEG, c)
        return c, acts, poss

    _, acts, poss = lax.fori_loop(
        0, K, step,
        (c_ref[...], jnp.zeros((BBLK, PADW), jnp.float32),
         jnp.zeros((BBLK, PADW), jnp.int32)))
    act_ref[...] = acts
    pos_ref[...] = poss


def _topcand(cand):
    return pl.pallas_call(
        _topcand_body,
        grid=(B // BBLK,),
        in_specs=[pl.BlockSpec((BBLK, NCAND), lambda i: (i, 0))],
        out_specs=[
            pl.BlockSpec((BBLK, PADW), lambda i: (i, 0)),
            pl.BlockSpec((BBLK, PADW), lambda i: (i, 0)),
        ],
        out_shape=[
            jax.ShapeDtypeStruct((B, PADW), jnp.float32),
            jax.ShapeDtypeStruct((B, PADW), jnp.int32),
        ],
    )(cand)


def _decode(W_dec, b_dec, seg_idx, acts, pos):
    mesh = plsc.VectorSubcoreMesh(core_axis_name="c", subcore_axis_name="s")
    info = plsc.get_sparse_core_info()
    nw = info.num_cores * info.num_subcores
    rows_per_w = B // nw
    HK = 16

    @functools.partial(
        pl.kernel,
        mesh=mesh,
        out_type=jax.ShapeDtypeStruct((B, D_IN), jnp.float32),
        scratch_types=[
            pltpu.VMEM((rows_per_w * PADW,), jnp.int32),
            pltpu.VMEM((rows_per_w * PADW,), jnp.int32),
            pltpu.VMEM((rows_per_w * PADW,), jnp.float32),
            pltpu.VMEM((D_IN,), jnp.float32),
            pltpu.VMEM((D_IN,), jnp.float32),
            pltpu.VMEM((HK, D_IN), jnp.float32),
            pltpu.VMEM((HK, D_IN), jnp.float32),
            pltpu.SemaphoreType.DMA,
            pltpu.SemaphoreType.DMA,
        ],
    )
    def k5(wdec_hbm, bdec_hbm, seg_hbm, act_hbm, pos_hbm, out_hbm,
           segs_v, pos_v, acts_v, bdec_v, acc_v, buf_a, buf_b, sem_a, sem_b):
        wid = lax.axis_index("s") * info.num_cores + lax.axis_index("c")
        base = wid * rows_per_w
        nflat = rows_per_w * PADW
        pltpu.sync_copy(seg_hbm.at[pl.ds(base * PADW, nflat)], segs_v)
        pltpu.sync_copy(pos_hbm.at[pl.ds(base * PADW, nflat)], pos_v)
        pltpu.sync_copy(act_hbm.at[pl.ds(base * PADW, nflat)], acts_v)
        pltpu.sync_copy(bdec_hbm, bdec_v)

        iota16 = lax.iota(jnp.int32, 16)

        def dyn_gather16(vec, idx):
            return lax.gather(
                vec, idx[:, None],
                lax.GatherDimensionNumbers(
                    offset_dims=(), collapsed_slice_dims=(0,),
                    start_index_map=(0,)),
                (1,), mode=lax.GatherScatterMode.PROMISE_IN_BOUNDS)

        def lat_for(r, h):
            p = pos_v[pl.ds(pl.multiple_of(r * PADW + h * 16, 16), 16)]
            sel = p >> 7
            seg_lo = segs_v[pl.ds(pl.multiple_of(r * PADW, 16), 16)]
            seg_hi = segs_v[pl.ds(pl.multiple_of(r * PADW + 16, 16), 16)]
            low = sel & 15
            g_lo = dyn_gather16(seg_lo, low)
            g_hi = dyn_gather16(seg_hi, low)
            s = jnp.where(sel < 16, g_lo, g_hi)
            return s * SEG + (p & 127)

        def accumulate(r, h, buf, init_from_bdec):
            av = acts_v[pl.ds(pl.multiple_of(r * PADW + h * 16, 16), 16)]
            a = [dyn_gather16(av, jnp.full((16,), k, jnp.int32))
                 for k in range(HK)]

            def chunk(c, carry):
                ds = pl.ds(pl.multiple_of(c * 16, 16), 16)
                if init_from_bdec:
                    base = bdec_v[ds]
                else:
                    base = acc_v[ds]
                v = [a[j] * buf[j, ds] for j in range(4)]
                for k in range(4, HK):
                    v[k % 4] = v[k % 4] + a[k] * buf[k, ds]
                acc_v[ds] = base + ((v[0] + v[1]) + (v[2] + v[3]))
                return carry

            lax.fori_loop(0, D_IN // 16, chunk, 0)

        def wait_a():
            pltpu.make_async_copy(wdec_hbm.at[pl.ds(0, HK)], buf_a,
                                  sem_a).wait()

        def row_body(r, carry):
            row = base + r
            cp_b = pltpu.async_copy(wdec_hbm.at[lat_for(r, 1)], buf_b, sem_b)
            wait_a()
            accumulate(r, 0, buf_a, True)
            r_next = jnp.minimum(r + 1, rows_per_w - 1)
            pltpu.async_copy(wdec_hbm.at[lat_for(r_next, 0)], buf_a, sem_a)
            cp_b.wait()
            accumulate(r, 1, buf_b, False)
            pltpu.sync_copy(acc_v, out_hbm.at[row])
            return carry

        pltpu.async_copy(wdec_hbm.at[lat_for(0, 0)], buf_a, sem_a)
        lax.fori_loop(0, rows_per_w, row_body, 0)
        wait_a()

    return k5(W_dec, b_dec, seg_idx, acts, pos)


def kernel(x, W_enc, b_enc, W_dec, b_dec):
    pre, M3 = _encode(x, W_enc, b_enc, b_dec)
    seg_idx = _topseg(M3.transpose(1, 0, 2).reshape(B, NSEG))
    cand = _gather_cand(pre.reshape(B * NSEG, SEG), seg_idx)
    acts, pos = _topcand(cand.reshape(B, NCAND))
    return _decode(W_dec, b_dec, seg_idx.reshape(-1), acts.reshape(-1),
                   pos.reshape(-1))


# speedup vs baseline: 5.4999x; 1.0811x over previous
import functools

import jax
import jax.numpy as jnp
from jax import lax
from jax.experimental import pallas as pl
from jax.experimental.pallas import tpu as pltpu
from jax.experimental.pallas import tpu_sc as plsc

D_IN = 2048
NL = 32768
K = 32
B = 1024

SEG = 128
NSEG = NL // SEG
LBLK = 1024
NLB = NL // LBLK
SEG_PER_BLK = LBLK // SEG
NCAND = K * SEG
PADW = 128

_NEG = -3.0e38


def _encode_body(x_ref, w_ref, benc_ref, bdec_ref, pre_ref, m_ref):
    xb = (x_ref[...] - bdec_ref[...]).astype(jnp.bfloat16)
    wb = w_ref[...].astype(jnp.bfloat16)
    pre = lax.dot_general(xb, wb, (((1,), (1,)), ((), ())),
                          preferred_element_type=jnp.float32)
    pre = pre + benc_ref[...].reshape(1, LBLK)
    pre_ref[...] = pre
    m_ref[...] = jnp.max(pre.reshape(B, SEG_PER_BLK, SEG), axis=-1)[None]


def _encode(x, W_enc, b_enc, b_dec):
    return pl.pallas_call(
        _encode_body,
        grid=(NLB,),
        in_specs=[
            pl.BlockSpec((B, D_IN), lambda i: (0, 0)),
            pl.BlockSpec((LBLK, D_IN), lambda i: (i, 0)),
            pl.BlockSpec((1, 1, LBLK), lambda i: (i, 0, 0)),
            pl.BlockSpec((1, D_IN), lambda i: (0, 0)),
        ],
        out_specs=[
            pl.BlockSpec((B, LBLK), lambda i: (0, i)),
            pl.BlockSpec((1, B, SEG_PER_BLK), lambda i: (i, 0, 0)),
        ],
        out_shape=[
            jax.ShapeDtypeStruct((B, NL), jnp.float32),
            jax.ShapeDtypeStruct((NLB, B, SEG_PER_BLK), jnp.float32),
        ],
    )(x, W_enc, b_enc.reshape(NLB, 1, LBLK), b_dec.reshape(1, D_IN))


def _topseg_body(m_ref, seg_ref):
    nb = m_ref.shape[0]
    iota = lax.broadcasted_iota(jnp.int32, (nb, NSEG), 1)
    lane = lax.broadcasted_iota(jnp.int32, (nb, PADW), 1)

    def step(k, carry):
        m, out = carry
        row_max = jnp.max(m, axis=1, keepdims=True)
        pos = jnp.min(jnp.where(m == row_max, iota, NSEG), axis=1,
                      keepdims=True)
        out = jnp.where(lane == k, pos, out)
        m = jnp.where(iota == pos, _NEG, m)
        return m, out

    _, out = lax.fori_loop(0, K, step,
                           (m_ref[...], jnp.zeros((nb, PADW), jnp.int32)))
    seg_ref[...] = out


def _topseg(M):
    nb = M.shape[0]
    return pl.pallas_call(
        _topseg_body,
        out_shape=jax.ShapeDtypeStruct((nb, PADW), jnp.int32),
    )(M)


def _gather_cand(pre_flat, seg_idx, row0, nb):
    mesh = plsc.VectorSubcoreMesh(core_axis_name="c", subcore_axis_name="s")
    info = plsc.get_sparse_core_info()
    nw = info.num_cores * info.num_subcores
    rows_per_w = nb // nw

    @functools.partial(
        pl.kernel,
        mesh=mesh,
        out_type=jax.ShapeDtypeStruct((nb, K, SEG), jnp.float32),
        scratch_types=[
            pltpu.VMEM((rows_per_w, PADW), jnp.int32),
            pltpu.VMEM((K,), jnp.int32),
            pltpu.VMEM((K, SEG), jnp.float32),
            pltpu.SemaphoreType.DMA,
        ],
    )
    def k3(pre_hbm, seg_hbm, cand_hbm, segs_v, idx_v, cand_v, sem):
        wid = lax.axis_index("s") * info.num_cores + lax.axis_index("c")
        base = wid * rows_per_w
        pltpu.sync_copy(seg_hbm.at[pl.ds(base, rows_per_w)], segs_v)

        def row_body(r, carry):
            row = base + r
            for c in range(K // 16):
                segs = segs_v[r, pl.ds(c * 16, 16)]
                idx_v[pl.ds(c * 16, 16)] = segs + (row0 + row) * NSEG
            pltpu.async_copy(pre_hbm.at[idx_v], cand_v, sem).wait()
            pltpu.sync_copy(cand_v, cand_hbm.at[row])
            return carry

        lax.fori_loop(0, rows_per_w, row_body, 0)

    return k3(pre_flat, seg_idx)


BBLK = 128


def _topcand_body(c_ref, act_ref, pos_ref):
    iota = lax.broadcasted_iota(jnp.int32, (BBLK, NCAND), 1)
    lane = lax.broadcasted_iota(jnp.int32, (BBLK, PADW), 1)

    def step(k, carry):
        c, acts, poss = carry
        row_max = jnp.max(c, axis=1, keepdims=True)
        pos = jnp.min(jnp.where(c == row_max, iota, NCAND), axis=1,
                      keepdims=True)
        acts = jnp.where(lane == k, row_max, acts)
        poss = jnp.where(lane == k, pos, poss)
        c = jnp.where(iota == pos, _NEG, c)
        return c, acts, poss

    _, acts, poss = lax.fori_loop(
        0, K, step,
        (c_ref[...], jnp.zeros((BBLK, PADW), jnp.float32),
         jnp.zeros((BBLK, PADW), jnp.int32)))
    act_ref[...] = acts
    pos_ref[...] = poss


def _topcand(cand):
    nb = cand.shape[0]
    return pl.pallas_call(
        _topcand_body,
        grid=(nb // BBLK,),
        in_specs=[pl.BlockSpec((BBLK, NCAND), lambda i: (i, 0))],
        out_specs=[
            pl.BlockSpec((BBLK, PADW), lambda i: (i, 0)),
            pl.BlockSpec((BBLK, PADW), lambda i: (i, 0)),
        ],
        out_shape=[
            jax.ShapeDtypeStruct((nb, PADW), jnp.float32),
            jax.ShapeDtypeStruct((nb, PADW), jnp.int32),
        ],
    )(cand)


def _decode(W_dec, b_dec, seg_idx, acts, pos, nb):
    mesh = plsc.VectorSubcoreMesh(core_axis_name="c", subcore_axis_name="s")
    info = plsc.get_sparse_core_info()
    nw = info.num_cores * info.num_subcores
    rows_per_w = nb // nw
    HK = 16

    @functools.partial(
        pl.kernel,
        mesh=mesh,
        out_type=jax.ShapeDtypeStruct((nb, D_IN), jnp.float32),
        scratch_types=[
            pltpu.VMEM((rows_per_w * PADW,), jnp.int32),
            pltpu.VMEM((rows_per_w * PADW,), jnp.int32),
            pltpu.VMEM((rows_per_w * PADW,), jnp.float32),
            pltpu.VMEM((D_IN,), jnp.float32),
            pltpu.VMEM((D_IN,), jnp.float32),
            pltpu.VMEM((HK, D_IN), jnp.float32),
            pltpu.VMEM((HK, D_IN), jnp.float32),
            pltpu.SemaphoreType.DMA,
            pltpu.SemaphoreType.DMA,
        ],
    )
    def k5(wdec_hbm, bdec_hbm, seg_hbm, act_hbm, pos_hbm, out_hbm,
           segs_v, pos_v, acts_v, bdec_v, acc_v, buf_a, buf_b, sem_a, sem_b):
        wid = lax.axis_index("s") * info.num_cores + lax.axis_index("c")
        base = wid * rows_per_w
        nflat = rows_per_w * PADW
        pltpu.sync_copy(seg_hbm.at[pl.ds(base * PADW, nflat)], segs_v)
        pltpu.sync_copy(pos_hbm.at[pl.ds(base * PADW, nflat)], pos_v)
        pltpu.sync_copy(act_hbm.at[pl.ds(base * PADW, nflat)], acts_v)
        pltpu.sync_copy(bdec_hbm, bdec_v)

        iota16 = lax.iota(jnp.int32, 16)

        def dyn_gather16(vec, idx):
            return lax.gather(
                vec, idx[:, None],
                lax.GatherDimensionNumbers(
                    offset_dims=(), collapsed_slice_dims=(0,),
                    start_index_map=(0,)),
                (1,), mode=lax.GatherScatterMode.PROMISE_IN_BOUNDS)

        def lat_for(r, h):
            p = pos_v[pl.ds(pl.multiple_of(r * PADW + h * 16, 16), 16)]
            sel = p >> 7
            seg_lo = segs_v[pl.ds(pl.multiple_of(r * PADW, 16), 16)]
            seg_hi = segs_v[pl.ds(pl.multiple_of(r * PADW + 16, 16), 16)]
            low = sel & 15
            g_lo = dyn_gather16(seg_lo, low)
            g_hi = dyn_gather16(seg_hi, low)
            s = jnp.where(sel < 16, g_lo, g_hi)
            return s * SEG + (p & 127)

        def accumulate(r, h, buf, init_from_bdec):
            av = acts_v[pl.ds(pl.multiple_of(r * PADW + h * 16, 16), 16)]
            a = [dyn_gather16(av, jnp.full((16,), k, jnp.int32))
                 for k in range(HK)]

            def chunk(c, carry):
                ds = pl.ds(pl.multiple_of(c * 16, 16), 16)
                if init_from_bdec:
                    base = bdec_v[ds]
                else:
                    base = acc_v[ds]
                v = [a[j] * buf[j, ds] for j in range(4)]
                for k in range(4, HK):
                    v[k % 4] = v[k % 4] + a[k] * buf[k, ds]
                acc_v[ds] = base + ((v[0] + v[1]) + (v[2] + v[3]))
                return carry

            lax.fori_loop(0, D_IN // 16, chunk, 0)

        def wait_a():
            pltpu.make_async_copy(wdec_hbm.at[pl.ds(0, HK)], buf_a,
                                  sem_a).wait()

        def row_body(r, carry):
            row = base + r
            cp_b = pltpu.async_copy(wdec_hbm.at[lat_for(r, 1)], buf_b, sem_b)
            wait_a()
            accumulate(r, 0, buf_a, True)
            r_next = jnp.minimum(r + 1, rows_per_w - 1)
            pltpu.async_copy(wdec_hbm.at[lat_for(r_next, 0)], buf_a, sem_a)
            cp_b.wait()
            accumulate(r, 1, buf_b, False)
            pltpu.sync_copy(acc_v, out_hbm.at[row])
            return carry

        pltpu.async_copy(wdec_hbm.at[lat_for(0, 0)], buf_a, sem_a)
        lax.fori_loop(0, rows_per_w, row_body, 0)
        wait_a()

    return k5(W_dec, b_dec, seg_idx, acts, pos)


def kernel(x, W_enc, b_enc, W_dec, b_dec):
    pre, M3 = _encode(x, W_enc, b_enc, b_dec)
    M = M3.transpose(1, 0, 2).reshape(B, NSEG)
    pre_flat = pre.reshape(B * NSEG, SEG)
    HB = B // 2
    outs = []
    for h in range(2):
        Mh = lax.slice(M, (h * HB, 0), ((h + 1) * HB, NSEG))
        seg_idx = _topseg(Mh)
        cand = _gather_cand(pre_flat, seg_idx, h * HB, HB)
        acts, pos = _topcand(cand.reshape(HB, NCAND))
        outs.append(_decode(W_dec, b_dec, seg_idx.reshape(-1),
                            acts.reshape(-1), pos.reshape(-1), HB))
    return jnp.concatenate(outs, axis=0)


# speedup vs baseline: 5.6929x; 1.0351x over previous
import functools

import jax
import jax.numpy as jnp
from jax import lax
from jax.experimental import pallas as pl
from jax.experimental.pallas import tpu as pltpu
from jax.experimental.pallas import tpu_sc as plsc

D_IN = 2048
NL = 32768
K = 32
B = 1024

SEG = 128
NSEG = NL // SEG
LBLK = 1024
NLB = NL // LBLK
SEG_PER_BLK = LBLK // SEG
NCAND = K * SEG
PADW = 128

_NEG = -3.0e38


def _encode_body(x_ref, w_ref, benc_ref, bdec_ref, pre_ref, m_ref):
    xb = (x_ref[...] - bdec_ref[...]).astype(jnp.bfloat16)
    wb = w_ref[...].astype(jnp.bfloat16)
    pre = lax.dot_general(xb, wb, (((1,), (1,)), ((), ())),
                          preferred_element_type=jnp.float32)
    pre = pre + benc_ref[...].reshape(1, LBLK)
    pre_ref[...] = pre
    m_ref[...] = jnp.max(pre.reshape(B, SEG_PER_BLK, SEG), axis=-1)[None]


def _encode(x, W_enc, b_enc, b_dec):
    return pl.pallas_call(
        _encode_body,
        grid=(NLB,),
        in_specs=[
            pl.BlockSpec((B, D_IN), lambda i: (0, 0)),
            pl.BlockSpec((LBLK, D_IN), lambda i: (i, 0)),
            pl.BlockSpec((1, 1, LBLK), lambda i: (i, 0, 0)),
            pl.BlockSpec((1, D_IN), lambda i: (0, 0)),
        ],
        out_specs=[
            pl.BlockSpec((B, LBLK), lambda i: (0, i)),
            pl.BlockSpec((1, B, SEG_PER_BLK), lambda i: (i, 0, 0)),
        ],
        out_shape=[
            jax.ShapeDtypeStruct((B, NL), jnp.float32),
            jax.ShapeDtypeStruct((NLB, B, SEG_PER_BLK), jnp.float32),
        ],
    )(x, W_enc, b_enc.reshape(NLB, 1, LBLK), b_dec.reshape(1, D_IN))


def _topseg_body(m_ref, seg_ref):
    nb = m_ref.shape[0]
    iota = lax.broadcasted_iota(jnp.int32, (nb, NSEG), 1)
    lane = lax.broadcasted_iota(jnp.int32, (nb, PADW), 1)

    def step(k, carry):
        m, out = carry
        row_max = jnp.max(m, axis=1, keepdims=True)
        pos = jnp.min(jnp.where(m == row_max, iota, NSEG), axis=1,
                      keepdims=True)
        out = jnp.where(lane == k, pos, out)
        m = jnp.where(iota == pos, _NEG, m)
        return m, out

    _, out = lax.fori_loop(0, K, step,
                           (m_ref[...], jnp.zeros((nb, PADW), jnp.int32)))
    seg_ref[...] = out


def _topseg(M):
    nb = M.shape[0]
    return pl.pallas_call(
        _topseg_body,
        out_shape=jax.ShapeDtypeStruct((nb, PADW), jnp.int32),
    )(M)


def _gather_cand(pre_flat, seg_idx, row0, nb):
    mesh = plsc.VectorSubcoreMesh(core_axis_name="c", subcore_axis_name="s")
    info = plsc.get_sparse_core_info()
    nw = info.num_cores * info.num_subcores
    rows_per_w = nb // nw

    @functools.partial(
        pl.kernel,
        mesh=mesh,
        out_type=jax.ShapeDtypeStruct((nb, K, SEG), jnp.float32),
        scratch_types=[
            pltpu.VMEM((rows_per_w, PADW), jnp.int32),
            pltpu.VMEM((K,), jnp.int32),
            pltpu.VMEM((K, SEG), jnp.float32),
            pltpu.SemaphoreType.DMA,
        ],
    )
    def k3(pre_hbm, seg_hbm, cand_hbm, segs_v, idx_v, cand_v, sem):
        wid = lax.axis_index("s") * info.num_cores + lax.axis_index("c")
        base = wid * rows_per_w
        pltpu.sync_copy(seg_hbm.at[pl.ds(base, rows_per_w)], segs_v)

        def row_body(r, carry):
            row = base + r
            for c in range(K // 16):
                segs = segs_v[r, pl.ds(c * 16, 16)]
                idx_v[pl.ds(c * 16, 16)] = segs + (row0 + row) * NSEG
            pltpu.async_copy(pre_hbm.at[idx_v], cand_v, sem).wait()
            pltpu.sync_copy(cand_v, cand_hbm.at[row])
            return carry

        lax.fori_loop(0, rows_per_w, row_body, 0)

    return k3(pre_flat, seg_idx)


BBLK = 128


def _topcand_body(c_ref, act_ref, pos_ref):
    iota = lax.broadcasted_iota(jnp.int32, (BBLK, NCAND), 1)
    lane = lax.broadcasted_iota(jnp.int32, (BBLK, PADW), 1)

    def step(k, carry):
        c, acts, poss = carry
        row_max = jnp.max(c, axis=1, keepdims=True)
        pos = jnp.min(jnp.where(c == row_max, iota, NCAND), axis=1,
                      keepdims=True)
        acts = jnp.where(lane == k, row_max, acts)
        poss = jnp.where(lane == k, pos, poss)
        c = jnp.where(iota == pos, _NEG, c)
        return c, acts, poss

    _, acts, poss = lax.fori_loop(
        0, K, step,
        (c_ref[...], jnp.zeros((BBLK, PADW), jnp.float32),
         jnp.zeros((BBLK, PADW), jnp.int32)))
    act_ref[...] = acts
    pos_ref[...] = poss


def _topcand(cand):
    nb = cand.shape[0]
    return pl.pallas_call(
        _topcand_body,
        grid=(nb // BBLK,),
        in_specs=[pl.BlockSpec((BBLK, NCAND), lambda i: (i, 0))],
        out_specs=[
            pl.BlockSpec((BBLK, PADW), lambda i: (i, 0)),
            pl.BlockSpec((BBLK, PADW), lambda i: (i, 0)),
        ],
        out_shape=[
            jax.ShapeDtypeStruct((nb, PADW), jnp.float32),
            jax.ShapeDtypeStruct((nb, PADW), jnp.int32),
        ],
    )(cand)


def _decode(W_dec, b_dec, seg_idx, acts, pos, nb):
    mesh = plsc.VectorSubcoreMesh(core_axis_name="c", subcore_axis_name="s")
    info = plsc.get_sparse_core_info()
    nw = info.num_cores * info.num_subcores
    rows_per_w = nb // nw
    HK = 16

    @functools.partial(
        pl.kernel,
        mesh=mesh,
        out_type=jax.ShapeDtypeStruct((nb, D_IN), jnp.float32),
        scratch_types=[
            pltpu.VMEM((rows_per_w * PADW,), jnp.int32),
            pltpu.VMEM((rows_per_w * PADW,), jnp.int32),
            pltpu.VMEM((rows_per_w * PADW,), jnp.float32),
            pltpu.VMEM((D_IN,), jnp.float32),
            pltpu.VMEM((D_IN,), jnp.float32),
            pltpu.VMEM((HK, D_IN), jnp.float32),
            pltpu.VMEM((HK, D_IN), jnp.float32),
            pltpu.SemaphoreType.DMA,
            pltpu.SemaphoreType.DMA,
        ],
    )
    def k5(wdec_hbm, bdec_hbm, seg_hbm, act_hbm, pos_hbm, out_hbm,
           segs_v, pos_v, acts_v, bdec_v, acc_v, buf_a, buf_b, sem_a, sem_b):
        wid = lax.axis_index("s") * info.num_cores + lax.axis_index("c")
        base = wid * rows_per_w
        nflat = rows_per_w * PADW
        pltpu.sync_copy(seg_hbm.at[pl.ds(base * PADW, nflat)], segs_v)
        pltpu.sync_copy(pos_hbm.at[pl.ds(base * PADW, nflat)], pos_v)
        pltpu.sync_copy(act_hbm.at[pl.ds(base * PADW, nflat)], acts_v)
        pltpu.sync_copy(bdec_hbm, bdec_v)

        iota16 = lax.iota(jnp.int32, 16)

        def dyn_gather16(vec, idx):
            return lax.gather(
                vec, idx[:, None],
                lax.GatherDimensionNumbers(
                    offset_dims=(), collapsed_slice_dims=(0,),
                    start_index_map=(0,)),
                (1,), mode=lax.GatherScatterMode.PROMISE_IN_BOUNDS)

        def lat_for(r, h):
            p = pos_v[pl.ds(pl.multiple_of(r * PADW + h * 16, 16), 16)]
            sel = p >> 7
            seg_lo = segs_v[pl.ds(pl.multiple_of(r * PADW, 16), 16)]
            seg_hi = segs_v[pl.ds(pl.multiple_of(r * PADW + 16, 16), 16)]
            low = sel & 15
            g_lo = dyn_gather16(seg_lo, low)
            g_hi = dyn_gather16(seg_hi, low)
            s = jnp.where(sel < 16, g_lo, g_hi)
            return s * SEG + (p & 127)

        def accumulate(r, h, buf, init_from_bdec):
            av = acts_v[pl.ds(pl.multiple_of(r * PADW + h * 16, 16), 16)]
            a = [dyn_gather16(av, jnp.full((16,), k, jnp.int32))
                 for k in range(HK)]

            def chunk(c, carry):
                ds = pl.ds(pl.multiple_of(c * 16, 16), 16)
                if init_from_bdec:
                    base = bdec_v[ds]
                else:
                    base = acc_v[ds]
                v = [a[j] * buf[j, ds] for j in range(4)]
                for k in range(4, HK):
                    v[k % 4] = v[k % 4] + a[k] * buf[k, ds]
                acc_v[ds] = base + ((v[0] + v[1]) + (v[2] + v[3]))
                return carry

            lax.fori_loop(0, D_IN // 16, chunk, 0)

        def wait_a():
            pltpu.make_async_copy(wdec_hbm.at[pl.ds(0, HK)], buf_a,
                                  sem_a).wait()

        def row_body(r, carry):
            row = base + r
            cp_b = pltpu.async_copy(wdec_hbm.at[lat_for(r, 1)], buf_b, sem_b)
            wait_a()
            accumulate(r, 0, buf_a, True)
            r_next = jnp.minimum(r + 1, rows_per_w - 1)
            pltpu.async_copy(wdec_hbm.at[lat_for(r_next, 0)], buf_a, sem_a)
            cp_b.wait()
            accumulate(r, 1, buf_b, False)
            pltpu.sync_copy(acc_v, out_hbm.at[row])
            return carry

        pltpu.async_copy(wdec_hbm.at[lat_for(0, 0)], buf_a, sem_a)
        lax.fori_loop(0, rows_per_w, row_body, 0)
        wait_a()

    return k5(W_dec, b_dec, seg_idx, acts, pos)


def kernel(x, W_enc, b_enc, W_dec, b_dec):
    pre, M3 = _encode(x, W_enc, b_enc, b_dec)
    M = M3.transpose(1, 0, 2).reshape(B, NSEG)
    pre_flat = pre.reshape(B * NSEG, SEG)
    NCHAIN = 4
    HB = B // NCHAIN
    outs = []
    for h in range(NCHAIN):
        Mh = lax.slice(M, (h * HB, 0), ((h + 1) * HB, NSEG))
        seg_idx = _topseg(Mh)
        cand = _gather_cand(pre_flat, seg_idx, h * HB, HB)
        acts, pos = _topcand(cand.reshape(HB, NCAND))
        outs.append(_decode(W_dec, b_dec, seg_idx.reshape(-1),
                            acts.reshape(-1), pos.reshape(-1), HB))
    return jnp.concatenate(outs, axis=0)
